# baseline (speedup 1.0000x reference)
import functools

import jax
import jax.numpy as jnp
from jax import lax
from jax.experimental import pallas as pl
from jax.experimental.pallas import tpu as pltpu
from jax.experimental.pallas import tpu_sc as plsc

_INV2 = 1.0 / 2.0 ** 0.5
_INV3 = 1.0 / 3.0 ** 0.5

_NC, _NS = 2, 16
_NW = _NC * _NS
_CH = 80
_K = 5


def _silu(x):
    return x * jax.nn.sigmoid(x)


def _dot(a, b):
    return jnp.dot(a, b, preferred_element_type=jnp.float32)


def _full(w):
    return pl.BlockSpec(w.shape, lambda i: (0,) * w.ndim)


def _phase_a(gall, de, p, tile):
    E = gall.shape[0]
    grid = (E // tile,)
    inv_nb = _INV2
    bd = jax.scipy.linalg.block_diag

    w1 = jnp.concatenate([p['q_W_m_rbf'], p['t_W_m_rbf'], p['W_ij']], axis=1)
    g1w = bd(p['q_W_rbf'], p['q_W_rbf'], p['t_W_rbf'], p['t_W_rbf'])
    m2w = bd(p['q_W_m_cbf'], p['q_W_m_cbf'], p['t_W_m_cbf'], p['t_W_m_cbf'])
    g2w = bd(p['q_W_cbf'], p['q_W_cbf'], p['t_W_cbf'], p['t_W_cbf'])
    m3w = bd(p['q_W_m_sbf'], p['q_W_m_sbf'])
    g3w = bd(p['q_W_sbf'], p['q_W_sbf'])
    dirw = bd(p['q_W_dir'], p['t_W_dir'])
    outw = bd(jnp.concatenate([p['q_W_out_ij'], p['q_W_out_ji']], axis=1),
              jnp.concatenate([p['t_W_out_ij'], p['t_W_out_ji']], axis=1))

    def body(g_ref, w1r, g1r, m2r, g2r, m3r, g3r, dirr, outr, awrr,
             a_out, b_out):
        g = g_ref[...]
        m = g[:, :de]
        t0 = _dot(m, w1r[...])
        tq = _silu(t0[:, :de])
        tt = _silu(t0[:, de:2 * de])
        base = t0[:, 2 * de:]
        u = jnp.concatenate([tq, tq, tt, tt], axis=1) \
            * _dot(g[:, de:2 * de], g1r[...])
        u = _silu(_dot(u, m2r[...]))
        u = u * _dot(g[:, 2 * de:3 * de], g2r[...])
        xt = (u[:, 2 * de:3 * de] + u[:, 3 * de:]) * inv_nb
        v = _silu(_dot(u[:, :2 * de], m3r[...]))
        v = v * _dot(g[:, 3 * de:4 * de], g3r[...])
        xq = (v[:, :de] + v[:, de:]) * inv_nb
        y = _silu(_dot(jnp.concatenate([xq, xt], axis=1), dirr[...]))
        z = _silu(_dot(y, outr[...]))
        rh_gate = _dot(g[:, 4 * de:], awrr[...])
        a_out[...] = jnp.concatenate(
            [base + _INV2 * (z[:, :de] + z[:, 2 * de:3 * de]), rh_gate], axis=1)
        b_out[...] = jnp.concatenate([z[:, de:2 * de], z[:, 3 * de:]], axis=1)

    ws = [w1, g1w, m2w, g2w, m3w, g3w, dirw, outw, p['a_W_rbf']]
    wide = gall.shape[1]
    return pl.pallas_call(
        body,
        grid=grid,
        in_specs=[pl.BlockSpec((tile, wide), lambda i: (i, 0))]
                 + [_full(w) for w in ws],
        out_specs=[pl.BlockSpec((tile, 2 * de), lambda i: (i, 0)),
                   pl.BlockSpec((tile, 2 * de), lambda i: (i, 0))],
        out_shape=[jax.ShapeDtypeStruct((E, 2 * de), jnp.float32),
                   jax.ShapeDtypeStruct((E, 2 * de), jnp.float32)],
    )(gall, *ws)


def _phase_c(a2, bg, gall, de, p, tile):
    E = a2.shape[0]
    grid = (E // tile,)

    def body(a_ref, bg_ref, m_ref, bs1, bs2, as1, as2, mx_out):
        a2v = a_ref[...]
        bg = bg_ref[...]
        x = (a2v[:, :de] + _INV2 * (bg[:, :de] + bg[:, de:])) * _INV3
        y = _silu(_dot(x, bs1[...]))
        y = _silu(_dot(y, bs2[...]))
        x = (x + y) * _INV2
        mm = (m_ref[:, :de] + x) * _INV2
        y = _silu(_dot(mm, as1[...]))
        y = _silu(_dot(y, as2[...]))
        mm = (mm + y) * _INV2
        mx_out[...] = jnp.concatenate([mm, mm * a2v[:, de:]], axis=1)

    ws = [p['bs_W1'], p['bs_W2'], p['as_W1'], p['as_W2']]
    spec2 = pl.BlockSpec((tile, 2 * de), lambda i: (i, 0))
    return pl.pallas_call(
        body,
        grid=grid,
        in_specs=[spec2, spec2, spec2] + [_full(w) for w in ws],
        out_specs=spec2,
        out_shape=jax.ShapeDtypeStruct((E, 2 * de), jnp.float32),
    )(a2, bg, gall, *ws)


def _phase_d(parts, h, p, tile):
    N, da = h.shape
    de = parts.shape[2] // 2
    grid = (N // tile,)

    def body(p_ref, h_ref, awd, ar1, ar2, h_out):
        seg = p_ref[0, :, de:] + p_ref[1, :, de:]
        xa = _silu(_dot(seg, awd[...]))
        y = _silu(_dot(xa, ar1[...]))
        y = _silu(_dot(y, ar2[...]))
        xa = (xa + y) * _INV2
        h_out[...] = (h_ref[...] + xa) * _INV2

    ws = [p['a_W_dense'], p['a_res_W1'], p['a_res_W2']]
    return pl.pallas_call(
        body,
        grid=grid,
        in_specs=[pl.BlockSpec((2, tile, 2 * de), lambda i: (0, i, 0)),
                  pl.BlockSpec((tile, da), lambda i: (i, 0))]
                 + [_full(w) for w in ws],
        out_specs=pl.BlockSpec((tile, da), lambda i: (i, 0)),
        out_shape=jax.ShapeDtypeStruct((N, da), jnp.float32),
    )(parts, h, *ws)


def _phase_e(mx, hi, hj, p, tile):
    E = mx.shape[0]
    de = mx.shape[1] // 2
    da = hi.shape[1]
    s_w = p['s_W']
    swi, swj, swm = s_w[:da], s_w[da:2 * da], s_w[2 * da:]
    grid = (E // tile,)

    def body(mx_ref, hi_ref, hj_ref, swi_r, swj_r, swm_r, aa1, aa2, out):
        mm = mx_ref[:, :de]
        m2 = _silu(_dot(hi_ref[...], swi_r[...]) + _dot(hj_ref[...], swj_r[...])
                   + _dot(mm, swm_r[...]))
        y = _silu(_dot(m2, aa1[...]))
        y = _silu(_dot(y, aa2[...]))
        m2 = (m2 + y) * _INV2
        out[...] = (mm + m2) * _INV2

    ws = [swi, swj, swm, p['aa_W1'], p['aa_W2']]
    spec = pl.BlockSpec((tile, de), lambda i: (i, 0))
    spec2 = pl.BlockSpec((tile, 2 * de), lambda i: (i, 0))
    speca = pl.BlockSpec((tile, da), lambda i: (i, 0))
    return pl.pallas_call(
        body,
        grid=grid,
        in_specs=[spec2, speca, speca] + [_full(w) for w in ws],
        out_specs=spec,
        out_shape=jax.ShapeDtypeStruct((E, de), jnp.float32),
    )(mx, hi, hj, *ws)


def _sc_gather_multi(tables, idx2ds):
    n = len(tables)
    w = tables[0].shape[1]
    nw, rows_w, ch = idx2ds[0].shape
    e_total = nw * rows_w * ch
    n_out = rows_w // _K
    mesh = plsc.VectorSubcoreMesh(core_axis_name="c", subcore_axis_name="s")

    @functools.partial(
        pl.kernel, mesh=mesh,
        out_type=[jax.ShapeDtypeStruct((e_total, w), jnp.float32)] * n,
        scratch_types=[pltpu.VMEM((rows_w, ch), jnp.int32)] * n
                      + [pltpu.VMEM((_K * ch, w), jnp.float32)] * n
                      + [pltpu.SemaphoreType.DMA, pltpu.SemaphoreType.DMA],
    )
    def k(*refs):
        tabs = refs[:n]
        idxs = refs[n:2 * n]
        outs = refs[2 * n:3 * n]
        idx_vs = refs[3 * n:4 * n]
        row_vs = refs[4 * n:5 * n]
        gsem, ssem = refs[5 * n:5 * n + 2]
        c = lax.axis_index("c")
        s = lax.axis_index("s")
        wid = s * _NC + c
        base = wid * rows_w * ch
        for t in range(n):
            pltpu.sync_copy(idxs[t].at[wid], idx_vs[t])

        def outer(o, carry):
            cps = []
            for t in range(n):
                for j in range(_K):
                    cps.append(pltpu.async_copy(
                        tabs[t].at[idx_vs[t].at[o * _K + j]],
                        row_vs[t].at[pl.ds(j * ch, ch)], gsem))
            for cp in cps:
                cp.wait()
            sts = []
            for t in range(n):
                sts.append(pltpu.async_copy(
                    row_vs[t], outs[t].at[pl.ds(base + o * _K * ch, _K * ch)],
                    ssem))
            for st in sts:
                st.wait()
            return carry

        lax.fori_loop(0, n_out, outer, 0)

    return k(*tables, *idx2ds)


def _sc_segment_sum(xa, idx2d, n_seg):
    e_total, w = xa.shape
    nw, rows_w, ch = idx2d.shape
    n_pair = (rows_w - 1) // 2
    n_init = 10
    rps = n_seg // n_init
    zch = 40
    nzch = rps // zch
    mesh = plsc.VectorSubcoreMesh(core_axis_name="c", subcore_axis_name="s")

    @functools.partial(
        pl.kernel, mesh=mesh,
        out_type=jax.ShapeDtypeStruct((_NC, n_seg, w), jnp.float32),
        scratch_types=[
            pltpu.VMEM((rows_w, ch), jnp.int32),
            pltpu.VMEM((ch, w), jnp.float32),
            pltpu.VMEM((ch, w), jnp.float32),
            pltpu.VMEM((zch, w), jnp.float32),
            pltpu.VMEM_SHARED((n_seg, w), jnp.float32),
            pltpu.SemaphoreType.DMA,
            pltpu.SemaphoreType.DMA,
            pltpu.SemaphoreType.DMA,
        ],
    )
    def k(xa_hbm, idx_hbm, out_hbm, idx_v, buf_a, buf_b, zb_v, acc,
          lsem_a, lsem_b, ssem):
        c = lax.axis_index("c")
        s = lax.axis_index("s")
        wid = s * _NC + c
        base = wid * rows_w * ch

        def zrow(r, carry):
            for q in range(w // 16):
                zb_v[r, pl.ds(q * 16, 16)] = jnp.zeros((16,), jnp.float32)
            return carry
        lax.fori_loop(0, zch, zrow, 0)

        @pl.when(s < n_init)
        def _():
            for t in range(nzch):
                pltpu.sync_copy(zb_v, acc.at[pl.ds(s * rps + t * zch, zch)])
        pltpu.sync_copy(idx_hbm.at[wid], idx_v)
        plsc.subcore_barrier()

        def load(o, buf, sem):
            pltpu.async_copy(xa_hbm.at[pl.ds(base + o * ch, ch)], buf, sem)

        def wait_load(o, buf, sem):
            pltpu.make_async_copy(
                xa_hbm.at[pl.ds(base + o * ch, ch)], buf, sem).wait()

        def scat(o, buf):
            pltpu.async_copy(buf, acc.at[idx_v.at[o]], ssem, add=True).wait()

        load(0, buf_a, lsem_a)

        def outer(t, carry):
            o = 2 * t
            load(o + 1, buf_b, lsem_b)
            wait_load(o, buf_a, lsem_a)
            scat(o, buf_a)
            load(o + 2, buf_a, lsem_a)
            wait_load(o + 1, buf_b, lsem_b)
            scat(o + 1, buf_b)
            return carry

        lax.fori_loop(0, n_pair, outer, 0)
        wait_load(rows_w - 1, buf_a, lsem_a)
        scat(rows_w - 1, buf_a)
        plsc.subcore_barrier()

        @pl.when(s < n_init)
        def _():
            for t in range(nzch):
                pltpu.sync_copy(acc.at[pl.ds(s * rps + t * zch, zch)], zb_v)
                pltpu.sync_copy(zb_v, out_hbm.at[c, pl.ds(s * rps + t * zch, zch)])

    return k(xa, idx2d)


def kernel(h, m_ij, rbf4, cbf4, sbf4, rbf3, cbf3, rbf_h, idx_i, idx_j,
           idx_swap, params):
    p = params
    n_nodes = h.shape[0]
    de = m_ij.shape[1]

    gall = jnp.concatenate(
        [m_ij, rbf4[0], rbf4[1], rbf3[0], rbf3[1],
         cbf4[0], cbf4[1], cbf3[0], cbf3[1], sbf4[0], sbf4[1], rbf_h], axis=1)

    a2, b = _phase_a(gall, de, p, tile=1000)
    (bg,) = _sc_gather_multi([b], [idx_swap.reshape(_NW, -1, _CH)])
    mx = _phase_c(a2, bg, gall, de, p, tile=1000)
    parts = _sc_segment_sum(mx, idx_i.reshape(_NW, -1, _CH), n_nodes)
    h_new = _phase_d(parts, h, p, tile=2000)
    hi, hj = _sc_gather_multi([h_new, h_new],
                              [idx_i.reshape(_NW, -1, 40),
                               idx_j.reshape(_NW, -1, 40)])
    m_new = _phase_e(mx, hi, hj, p, tile=1000)
    return h_new, m_new


# speedup vs baseline: 1.0665x; 1.0665x over previous
import functools

import jax
import jax.numpy as jnp
from jax import lax
from jax.experimental import pallas as pl
from jax.experimental.pallas import tpu as pltpu
from jax.experimental.pallas import tpu_sc as plsc

_INV2 = 1.0 / 2.0 ** 0.5
_INV3 = 1.0 / 3.0 ** 0.5

_NC, _NS = 2, 16
_NW = _NC * _NS
_CH = 80
_K = 5


def _silu(x):
    return x * jax.nn.sigmoid(x)


def _dot(a, b):
    return jnp.dot(a, b, preferred_element_type=jnp.float32)


def _full(w):
    return pl.BlockSpec(w.shape, lambda i: (0,) * w.ndim)


def _phase_a(m_ij, r4v, c4v, s4av, s4bv, r3v, c3v, rhv, p, tile):
    E, de = m_ij.shape
    grid = (E // tile,)
    inv_nb = _INV2
    bd = jax.scipy.linalg.block_diag

    t8 = lambda w: jnp.tile(w, (8, 1))
    w1 = jnp.concatenate([p['q_W_m_rbf'], p['t_W_m_rbf'], p['W_ij']], axis=1)
    m2w = bd(p['q_W_m_cbf'], p['q_W_m_cbf'], p['t_W_m_cbf'], p['t_W_m_cbf'])
    m3w = bd(p['q_W_m_sbf'], p['q_W_m_sbf'])
    dirw = bd(p['q_W_dir'], p['t_W_dir'])
    outw = bd(jnp.concatenate([p['q_W_out_ij'], p['q_W_out_ji']], axis=1),
              jnp.concatenate([p['t_W_out_ij'], p['t_W_out_ji']], axis=1))
    t8qr, t8tr = t8(p['q_W_rbf']), t8(p['t_W_rbf'])
    t8qc, t8tc = t8(p['q_W_cbf']), t8(p['t_W_cbf'])
    t8qsa, t8qsb = t8(p['q_W_sbf'][:16]), t8(p['q_W_sbf'][16:])
    t8awr = t8(p['a_W_rbf'])

    def body(m_ref, r4_ref, c4_ref, s4a_ref, s4b_ref, r3_ref, c3_ref, rh_ref,
             w1r, m2r, m3r, dirr, outr,
             qrr, trr, qcr, tcr, qsar, qsbr, awrr,
             a_out, b_out):
        m = m_ref[...]
        row_id = lax.broadcasted_iota(jnp.int32, (tile, 128), 0)
        lane_id = lax.broadcasted_iota(jnp.int32, (tile, 128), 1)
        own = (lane_id // 16) == (row_id % 8)
        zero = jnp.zeros((tile, 128), jnp.float32)

        def sel(pk):
            rows = pk.shape[0]
            e8 = jnp.broadcast_to(pk[:, None, :], (rows, 8, 128))
            return jnp.where(own, e8.reshape(rows * 8, 128), zero)

        def gate(pk, wr):
            return _dot(sel(pk), wr[...])

        t0 = _dot(m, w1r[...])
        tq = _silu(t0[:, :de])
        tt = _silu(t0[:, de:2 * de])
        base = t0[:, 2 * de:]
        g1 = jnp.concatenate(
            [gate(r4_ref[0], qrr), gate(r4_ref[1], qrr),
             gate(r3_ref[0], trr), gate(r3_ref[1], trr)], axis=1)
        u = jnp.concatenate([tq, tq, tt, tt], axis=1) * g1
        u = _silu(_dot(u, m2r[...]))
        g2 = jnp.concatenate(
            [gate(c4_ref[0], qcr), gate(c4_ref[1], qcr),
             gate(c3_ref[0], tcr), gate(c3_ref[1], tcr)], axis=1)
        u = u * g2
        xt = (u[:, 2 * de:3 * de] + u[:, 3 * de:]) * inv_nb
        v = _silu(_dot(u[:, :2 * de], m3r[...]))
        g3 = jnp.concatenate(
            [gate(s4a_ref[0], qsar) + gate(s4b_ref[0], qsbr),
             gate(s4a_ref[1], qsar) + gate(s4b_ref[1], qsbr)], axis=1)
        v = v * g3
        xq = (v[:, :de] + v[:, de:]) * inv_nb
        y = _silu(_dot(jnp.concatenate([xq, xt], axis=1), dirr[...]))
        z = _silu(_dot(y, outr[...]))
        rh_gate = gate(rh_ref[...], awrr)
        a_out[...] = jnp.concatenate(
            [base + _INV2 * (z[:, :de] + z[:, 2 * de:3 * de]), rh_gate], axis=1)
        b_out[...] = jnp.concatenate([z[:, de:2 * de], z[:, 3 * de:]], axis=1)

    ws = [w1, m2w, m3w, dirw, outw, t8qr, t8tr, t8qc, t8tc, t8qsa, t8qsb, t8awr]
    pk = lambda d: pl.BlockSpec((2, tile * d // 128, 128), lambda i: (0, i, 0))
    return pl.pallas_call(
        body,
        grid=grid,
        in_specs=[pl.BlockSpec((tile, de), lambda i: (i, 0)),
                  pk(16), pk(16), pk(16), pk(16), pk(16), pk(16),
                  pl.BlockSpec((tile * 16 // 128, 128), lambda i: (i, 0))]
                 + [_full(w) for w in ws],
        out_specs=[pl.BlockSpec((tile, 2 * de), lambda i: (i, 0)),
                   pl.BlockSpec((tile, 2 * de), lambda i: (i, 0))],
        out_shape=[jax.ShapeDtypeStruct((E, 2 * de), jnp.float32),
                   jax.ShapeDtypeStruct((E, 2 * de), jnp.float32)],
    )(m_ij, r4v, c4v, s4av, s4bv, r3v, c3v, rhv, *ws)


def _phase_c(a2, bg, m_ij, de, p, tile):
    E = a2.shape[0]
    grid = (E // tile,)

    def body(a_ref, bg_ref, m_ref, bs1, bs2, as1, as2, mx_out):
        a2v = a_ref[...]
        bg = bg_ref[...]
        x = (a2v[:, :de] + _INV2 * (bg[:, :de] + bg[:, de:])) * _INV3
        y = _silu(_dot(x, bs1[...]))
        y = _silu(_dot(y, bs2[...]))
        x = (x + y) * _INV2
        mm = (m_ref[...] + x) * _INV2
        y = _silu(_dot(mm, as1[...]))
        y = _silu(_dot(y, as2[...]))
        mm = (mm + y) * _INV2
        mx_out[...] = jnp.concatenate([mm, mm * a2v[:, de:]], axis=1)

    ws = [p['bs_W1'], p['bs_W2'], p['as_W1'], p['as_W2']]
    spec = pl.BlockSpec((tile, de), lambda i: (i, 0))
    spec2 = pl.BlockSpec((tile, 2 * de), lambda i: (i, 0))
    return pl.pallas_call(
        body,
        grid=grid,
        in_specs=[spec2, spec2, spec] + [_full(w) for w in ws],
        out_specs=spec2,
        out_shape=jax.ShapeDtypeStruct((E, 2 * de), jnp.float32),
    )(a2, bg, m_ij, *ws)


def _phase_d(parts, h, p, tile):
    N, da = h.shape
    de = parts.shape[2] // 2
    grid = (N // tile,)

    def body(p_ref, h_ref, awd, ar1, ar2, h_out):
        seg = p_ref[0, :, de:] + p_ref[1, :, de:]
        xa = _silu(_dot(seg, awd[...]))
        y = _silu(_dot(xa, ar1[...]))
        y = _silu(_dot(y, ar2[...]))
        xa = (xa + y) * _INV2
        h_out[...] = (h_ref[...] + xa) * _INV2

    ws = [p['a_W_dense'], p['a_res_W1'], p['a_res_W2']]
    return pl.pallas_call(
        body,
        grid=grid,
        in_specs=[pl.BlockSpec((2, tile, 2 * de), lambda i: (0, i, 0)),
                  pl.BlockSpec((tile, da), lambda i: (i, 0))]
                 + [_full(w) for w in ws],
        out_specs=pl.BlockSpec((tile, da), lambda i: (i, 0)),
        out_shape=jax.ShapeDtypeStruct((N, da), jnp.float32),
    )(parts, h, *ws)


def _phase_e(mx, hi, hj, p, tile):
    E = mx.shape[0]
    de = mx.shape[1] // 2
    da = hi.shape[1]
    s_w = p['s_W']
    swi, swj, swm = s_w[:da], s_w[da:2 * da], s_w[2 * da:]
    grid = (E // tile,)

    def body(mx_ref, hi_ref, hj_ref, swi_r, swj_r, swm_r, aa1, aa2, out):
        mm = mx_ref[:, :de]
        m2 = _silu(_dot(hi_ref[...], swi_r[...]) + _dot(hj_ref[...], swj_r[...])
                   + _dot(mm, swm_r[...]))
        y = _silu(_dot(m2, aa1[...]))
        y = _silu(_dot(y, aa2[...]))
        m2 = (m2 + y) * _INV2
        out[...] = (mm + m2) * _INV2

    ws = [swi, swj, swm, p['aa_W1'], p['aa_W2']]
    spec = pl.BlockSpec((tile, de), lambda i: (i, 0))
    spec2 = pl.BlockSpec((tile, 2 * de), lambda i: (i, 0))
    speca = pl.BlockSpec((tile, da), lambda i: (i, 0))
    return pl.pallas_call(
        body,
        grid=grid,
        in_specs=[spec2, speca, speca] + [_full(w) for w in ws],
        out_specs=spec,
        out_shape=jax.ShapeDtypeStruct((E, de), jnp.float32),
    )(mx, hi, hj, *ws)


def _sc_gather_multi(tables, idx2ds):
    n = len(tables)
    w = tables[0].shape[1]
    nw, rows_w, ch = idx2ds[0].shape
    e_total = nw * rows_w * ch
    n_out = rows_w // _K
    mesh = plsc.VectorSubcoreMesh(core_axis_name="c", subcore_axis_name="s")

    @functools.partial(
        pl.kernel, mesh=mesh,
        out_type=[jax.ShapeDtypeStruct((e_total, w), jnp.float32)] * n,
        scratch_types=[pltpu.VMEM((rows_w, ch), jnp.int32)] * n
                      + [pltpu.VMEM((_K * ch, w), jnp.float32)] * n
                      + [pltpu.SemaphoreType.DMA, pltpu.SemaphoreType.DMA],
    )
    def k(*refs):
        tabs = refs[:n]
        idxs = refs[n:2 * n]
        outs = refs[2 * n:3 * n]
        idx_vs = refs[3 * n:4 * n]
        row_vs = refs[4 * n:5 * n]
        gsem, ssem = refs[5 * n:5 * n + 2]
        c = lax.axis_index("c")
        s = lax.axis_index("s")
        wid = s * _NC + c
        base = wid * rows_w * ch
        for t in range(n):
            pltpu.sync_copy(idxs[t].at[wid], idx_vs[t])

        def outer(o, carry):
            cps = []
            for t in range(n):
                for j in range(_K):
                    cps.append(pltpu.async_copy(
                        tabs[t].at[idx_vs[t].at[o * _K + j]],
                        row_vs[t].at[pl.ds(j * ch, ch)], gsem))
            for cp in cps:
                cp.wait()
            sts = []
            for t in range(n):
                sts.append(pltpu.async_copy(
                    row_vs[t], outs[t].at[pl.ds(base + o * _K * ch, _K * ch)],
                    ssem))
            for st in sts:
                st.wait()
            return carry

        lax.fori_loop(0, n_out, outer, 0)

    return k(*tables, *idx2ds)


def _sc_segment_sum(xa, idx2d, n_seg):
    e_total, w = xa.shape
    nw, rows_w, ch = idx2d.shape
    n_pair = (rows_w - 1) // 2
    n_init = 10
    rps = n_seg // n_init
    zch = 40
    nzch = rps // zch
    mesh = plsc.VectorSubcoreMesh(core_axis_name="c", subcore_axis_name="s")

    @functools.partial(
        pl.kernel, mesh=mesh,
        out_type=jax.ShapeDtypeStruct((_NC, n_seg, w), jnp.float32),
        scratch_types=[
            pltpu.VMEM((rows_w, ch), jnp.int32),
            pltpu.VMEM((ch, w), jnp.float32),
            pltpu.VMEM((ch, w), jnp.float32),
            pltpu.VMEM((zch, w), jnp.float32),
            pltpu.VMEM_SHARED((n_seg, w), jnp.float32),
            pltpu.SemaphoreType.DMA,
            pltpu.SemaphoreType.DMA,
            pltpu.SemaphoreType.DMA,
        ],
    )
    def k(xa_hbm, idx_hbm, out_hbm, idx_v, buf_a, buf_b, zb_v, acc,
          lsem_a, lsem_b, ssem):
        c = lax.axis_index("c")
        s = lax.axis_index("s")
        wid = s * _NC + c
        base = wid * rows_w * ch

        def zrow(r, carry):
            for q in range(w // 16):
                zb_v[r, pl.ds(q * 16, 16)] = jnp.zeros((16,), jnp.float32)
            return carry
        lax.fori_loop(0, zch, zrow, 0)

        @pl.when(s < n_init)
        def _():
            for t in range(nzch):
                pltpu.sync_copy(zb_v, acc.at[pl.ds(s * rps + t * zch, zch)])
        pltpu.sync_copy(idx_hbm.at[wid], idx_v)
        plsc.subcore_barrier()

        def load(o, buf, sem):
            pltpu.async_copy(xa_hbm.at[pl.ds(base + o * ch, ch)], buf, sem)

        def wait_load(o, buf, sem):
            pltpu.make_async_copy(
                xa_hbm.at[pl.ds(base + o * ch, ch)], buf, sem).wait()

        def scat(o, buf):
            pltpu.async_copy(buf, acc.at[idx_v.at[o]], ssem, add=True).wait()

        load(0, buf_a, lsem_a)

        def outer(t, carry):
            o = 2 * t
            load(o + 1, buf_b, lsem_b)
            wait_load(o, buf_a, lsem_a)
            scat(o, buf_a)
            load(o + 2, buf_a, lsem_a)
            wait_load(o + 1, buf_b, lsem_b)
            scat(o + 1, buf_b)
            return carry

        lax.fori_loop(0, n_pair, outer, 0)
        wait_load(rows_w - 1, buf_a, lsem_a)
        scat(rows_w - 1, buf_a)
        plsc.subcore_barrier()

        @pl.when(s < n_init)
        def _():
            for t in range(nzch):
                pltpu.sync_copy(acc.at[pl.ds(s * rps + t * zch, zch)], zb_v)
                pltpu.sync_copy(zb_v, out_hbm.at[c, pl.ds(s * rps + t * zch, zch)])

    return k(xa, idx2d)


def kernel(h, m_ij, rbf4, cbf4, sbf4, rbf3, cbf3, rbf_h, idx_i, idx_j,
           idx_swap, params):
    p = params
    n_nodes = h.shape[0]
    de = m_ij.shape[1]

    e_edges = m_ij.shape[0]
    r4v = rbf4.reshape(2, -1, 128)
    c4v = cbf4.reshape(2, -1, 128)
    s4av = sbf4[:, :, :16].reshape(2, -1, 128)
    s4bv = sbf4[:, :, 16:].reshape(2, -1, 128)
    r3v = rbf3.reshape(2, -1, 128)
    c3v = cbf3.reshape(2, -1, 128)
    rhv = rbf_h.reshape(-1, 128)

    a2, b = _phase_a(m_ij, r4v, c4v, s4av, s4bv, r3v, c3v, rhv, p, tile=1600)
    (bg,) = _sc_gather_multi([b], [idx_swap.reshape(_NW, -1, _CH)])
    mx = _phase_c(a2, bg, m_ij, de, p, tile=1000)
    parts = _sc_segment_sum(mx, idx_i.reshape(_NW, -1, _CH), n_nodes)
    h_new = _phase_d(parts, h, p, tile=2000)
    hi, hj = _sc_gather_multi([h_new, h_new],
                              [idx_i.reshape(_NW, -1, 40),
                               idx_j.reshape(_NW, -1, 40)])
    m_new = _phase_e(mx, hi, hj, p, tile=1000)
    return h_new, m_new


# speedup vs baseline: 1.4485x; 1.3582x over previous
import functools

import jax
import jax.numpy as jnp
from jax import lax
from jax.experimental import pallas as pl
from jax.experimental.pallas import tpu as pltpu
from jax.experimental.pallas import tpu_sc as plsc

_INV2 = 1.0 / 2.0 ** 0.5
_INV3 = 1.0 / 3.0 ** 0.5

_NC, _NS = 2, 16
_NW = _NC * _NS
_CH = 80
_K = 5


def _silu(x):
    return x * jax.nn.sigmoid(x)


def _dot(a, b):
    return jnp.dot(a, b, preferred_element_type=jnp.float32)


def _full(w):
    return pl.BlockSpec(w.shape, lambda i: (0,) * w.ndim)


def _phase_a(m_ij, rbf4, cbf4, sbf4, rbf3, cbf3, rbf_h, p, tile):
    E, de = m_ij.shape
    grid = (E // tile,)
    inv_nb = _INV2
    bd = jax.scipy.linalg.block_diag

    w1 = jnp.concatenate([p['q_W_m_rbf'], p['t_W_m_rbf'], p['W_ij']], axis=1)
    g1w = bd(p['q_W_rbf'], p['q_W_rbf'], p['t_W_rbf'], p['t_W_rbf'])
    m2w = bd(p['q_W_m_cbf'], p['q_W_m_cbf'], p['t_W_m_cbf'], p['t_W_m_cbf'])
    g2w = bd(p['q_W_cbf'], p['q_W_cbf'], p['t_W_cbf'], p['t_W_cbf'])
    m3w = bd(p['q_W_m_sbf'], p['q_W_m_sbf'])
    g3w = bd(p['q_W_sbf'], p['q_W_sbf'])
    dirw = bd(p['q_W_dir'], p['t_W_dir'])
    outw = bd(jnp.concatenate([p['q_W_out_ij'], p['q_W_out_ji']], axis=1),
              jnp.concatenate([p['t_W_out_ij'], p['t_W_out_ji']], axis=1))

    def body(m_ref, r4_ref, c4_ref, s4_ref, r3_ref, c3_ref, rh_ref,
             w1r, g1r, m2r, g2r, m3r, g3r, dirr, outr, awrr,
             a_out, b_out):
        m = m_ref[...]
        t0 = _dot(m, w1r[...])
        tq = _silu(t0[:, :de])
        tt = _silu(t0[:, de:2 * de])
        base = t0[:, 2 * de:]
        g1in = jnp.concatenate([r4_ref[0], r4_ref[1], r3_ref[0], r3_ref[1]],
                               axis=1)
        u = jnp.concatenate([tq, tq, tt, tt], axis=1) * _dot(g1in, g1r[...])
        u = _silu(_dot(u, m2r[...]))
        g2in = jnp.concatenate([c4_ref[0], c4_ref[1], c3_ref[0], c3_ref[1]],
                               axis=1)
        u = u * _dot(g2in, g2r[...])
        xt = (u[:, 2 * de:3 * de] + u[:, 3 * de:]) * inv_nb
        v = _silu(_dot(u[:, :2 * de], m3r[...]))
        g3in = jnp.concatenate([s4_ref[0], s4_ref[1]], axis=1)
        v = v * _dot(g3in, g3r[...])
        xq = (v[:, :de] + v[:, de:]) * inv_nb
        y = _silu(_dot(jnp.concatenate([xq, xt], axis=1), dirr[...]))
        z = _silu(_dot(y, outr[...]))
        rh_gate = _dot(rh_ref[...], awrr[...])
        a_out[...] = jnp.concatenate(
            [base + _INV2 * (z[:, :de] + z[:, 2 * de:3 * de]), rh_gate], axis=1)
        b_out[...] = jnp.concatenate([z[:, de:2 * de], z[:, 3 * de:]], axis=1)

    ws = [w1, g1w, m2w, g2w, m3w, g3w, dirw, outw, p['a_W_rbf']]
    edge3 = lambda d: pl.BlockSpec((2, tile, d), lambda i: (0, i, 0))
    return pl.pallas_call(
        body,
        grid=grid,
        in_specs=[pl.BlockSpec((tile, de), lambda i: (i, 0)),
                  edge3(16), edge3(16), edge3(32), edge3(16), edge3(16),
                  pl.BlockSpec((tile, 16), lambda i: (i, 0))]
                 + [_full(w) for w in ws],
        out_specs=[pl.BlockSpec((tile, 2 * de), lambda i: (i, 0)),
                   pl.BlockSpec((tile, 2 * de), lambda i: (i, 0))],
        out_shape=[jax.ShapeDtypeStruct((E, 2 * de), jnp.float32),
                   jax.ShapeDtypeStruct((E, 2 * de), jnp.float32)],
    )(m_ij, rbf4, cbf4, sbf4, rbf3, cbf3, rbf_h, *ws)


def _phase_c(a2, bg, m_ij, de, p, tile, goff):
    eh = bg.shape[0]
    grid = (eh // tile,)

    def body(a_ref, bg_ref, m_ref, bs1, bs2, as1, as2, mx_out):
        a2v = a_ref[...]
        bg = bg_ref[...]
        x = (a2v[:, :de] + _INV2 * (bg[:, :de] + bg[:, de:])) * _INV3
        y = _silu(_dot(x, bs1[...]))
        y = _silu(_dot(y, bs2[...]))
        x = (x + y) * _INV2
        mm = (m_ref[...] + x) * _INV2
        y = _silu(_dot(mm, as1[...]))
        y = _silu(_dot(y, as2[...]))
        mm = (mm + y) * _INV2
        mx_out[...] = jnp.concatenate([mm, mm * a2v[:, de:]], axis=1)

    ws = [p['bs_W1'], p['bs_W2'], p['as_W1'], p['as_W2']]
    specf = pl.BlockSpec((tile, de), lambda i: (i + goff, 0))
    spec2f = pl.BlockSpec((tile, 2 * de), lambda i: (i + goff, 0))
    spec2 = pl.BlockSpec((tile, 2 * de), lambda i: (i, 0))
    return pl.pallas_call(
        body,
        grid=grid,
        in_specs=[spec2f, spec2, specf] + [_full(w) for w in ws],
        out_specs=spec2,
        out_shape=jax.ShapeDtypeStruct((eh, 2 * de), jnp.float32),
    )(a2, bg, m_ij, *ws)


def _phase_d(parts0, parts1, h, p, tile):
    N, da = h.shape
    de = parts0.shape[2] // 2
    grid = (N // tile,)

    def body(p0_ref, p1_ref, h_ref, awd, ar1, ar2, h_out):
        seg = (p0_ref[0, :, de:] + p0_ref[1, :, de:]
               + p1_ref[0, :, de:] + p1_ref[1, :, de:])
        xa = _silu(_dot(seg, awd[...]))
        y = _silu(_dot(xa, ar1[...]))
        y = _silu(_dot(y, ar2[...]))
        xa = (xa + y) * _INV2
        h_out[...] = (h_ref[...] + xa) * _INV2

    ws = [p['a_W_dense'], p['a_res_W1'], p['a_res_W2']]
    pspec = pl.BlockSpec((2, tile, 2 * de), lambda i: (0, i, 0))
    return pl.pallas_call(
        body,
        grid=grid,
        in_specs=[pspec, pspec,
                  pl.BlockSpec((tile, da), lambda i: (i, 0))]
                 + [_full(w) for w in ws],
        out_specs=pl.BlockSpec((tile, da), lambda i: (i, 0)),
        out_shape=jax.ShapeDtypeStruct((N, da), jnp.float32),
    )(parts0, parts1, h, *ws)


def _phase_e(mx, hi, hj, p, tile):
    E = hi.shape[0]
    de = mx.shape[1] // 2
    da = hi.shape[1]
    s_w = p['s_W']
    swi, swj, swm = s_w[:da], s_w[da:2 * da], s_w[2 * da:]
    grid = (E // tile,)

    def body(mx_ref, hi_ref, hj_ref, swi_r, swj_r, swm_r, aa1, aa2, out):
        mm = mx_ref[:, :de]
        m2 = _silu(_dot(hi_ref[...], swi_r[...]) + _dot(hj_ref[...], swj_r[...])
                   + _dot(mm, swm_r[...]))
        y = _silu(_dot(m2, aa1[...]))
        y = _silu(_dot(y, aa2[...]))
        m2 = (m2 + y) * _INV2
        out[...] = (mm + m2) * _INV2

    ws = [swi, swj, swm, p['aa_W1'], p['aa_W2']]
    spec = pl.BlockSpec((tile, de), lambda i: (i, 0))
    spec2 = pl.BlockSpec((tile, 2 * de), lambda i: (i, 0))
    speca = pl.BlockSpec((tile, da), lambda i: (i, 0))
    return pl.pallas_call(
        body,
        grid=grid,
        in_specs=[spec2, speca, speca] + [_full(w) for w in ws],
        out_specs=spec,
        out_shape=jax.ShapeDtypeStruct((E, de), jnp.float32),
    )(mx, hi, hj, *ws)


def _sc_gather_multi(tables, idx2ds):
    n = len(tables)
    w = tables[0].shape[1]
    nw, rows_w, ch = idx2ds[0].shape
    e_total = nw * rows_w * ch
    n_out = rows_w // _K
    mesh = plsc.VectorSubcoreMesh(core_axis_name="c", subcore_axis_name="s")

    @functools.partial(
        pl.kernel, mesh=mesh,
        out_type=[jax.ShapeDtypeStruct((e_total, w), jnp.float32)] * n,
        scratch_types=[pltpu.VMEM((rows_w, ch), jnp.int32)] * n
                      + [pltpu.VMEM((_K * ch, w), jnp.float32)] * n
                      + [pltpu.SemaphoreType.DMA, pltpu.SemaphoreType.DMA],
    )
    def k(*refs):
        tabs = refs[:n]
        idxs = refs[n:2 * n]
        outs = refs[2 * n:3 * n]
        idx_vs = refs[3 * n:4 * n]
        row_vs = refs[4 * n:5 * n]
        gsem, ssem = refs[5 * n:5 * n + 2]
        c = lax.axis_index("c")
        s = lax.axis_index("s")
        wid = s * _NC + c
        base = wid * rows_w * ch
        for t in range(n):
            pltpu.sync_copy(idxs[t].at[wid], idx_vs[t])

        def outer(o, carry):
            cps = []
            for t in range(n):
                for j in range(_K):
                    cps.append(pltpu.async_copy(
                        tabs[t].at[idx_vs[t].at[o * _K + j]],
                        row_vs[t].at[pl.ds(j * ch, ch)], gsem))
            for cp in cps:
                cp.wait()
            sts = []
            for t in range(n):
                sts.append(pltpu.async_copy(
                    row_vs[t], outs[t].at[pl.ds(base + o * _K * ch, _K * ch)],
                    ssem))
            for st in sts:
                st.wait()
            return carry

        lax.fori_loop(0, n_out, outer, 0)

    return k(*tables, *idx2ds)


def _sc_segment_sum(xa, idx2d, n_seg):
    e_total, w = xa.shape
    nw, rows_w, ch = idx2d.shape
    n_pair = (rows_w - 1) // 2
    n_init = 10
    rps = n_seg // n_init
    zch = 40
    nzch = rps // zch
    mesh = plsc.VectorSubcoreMesh(core_axis_name="c", subcore_axis_name="s")

    @functools.partial(
        pl.kernel, mesh=mesh,
        out_type=jax.ShapeDtypeStruct((_NC, n_seg, w), jnp.float32),
        scratch_types=[
            pltpu.VMEM((rows_w, ch), jnp.int32),
            pltpu.VMEM((ch, w), jnp.float32),
            pltpu.VMEM((ch, w), jnp.float32),
            pltpu.VMEM((zch, w), jnp.float32),
            pltpu.VMEM_SHARED((n_seg, w), jnp.float32),
            pltpu.SemaphoreType.DMA,
            pltpu.SemaphoreType.DMA,
            pltpu.SemaphoreType.DMA,
        ],
    )
    def k(xa_hbm, idx_hbm, out_hbm, idx_v, buf_a, buf_b, zb_v, acc,
          lsem_a, lsem_b, ssem):
        c = lax.axis_index("c")
        s = lax.axis_index("s")
        wid = s * _NC + c
        base = wid * rows_w * ch

        def zrow(r, carry):
            for q in range(w // 16):
                zb_v[r, pl.ds(q * 16, 16)] = jnp.zeros((16,), jnp.float32)
            return carry
        lax.fori_loop(0, zch, zrow, 0)

        @pl.when(s < n_init)
        def _():
            for t in range(nzch):
                pltpu.sync_copy(zb_v, acc.at[pl.ds(s * rps + t * zch, zch)])
        pltpu.sync_copy(idx_hbm.at[wid], idx_v)
        plsc.subcore_barrier()

        def load(o, buf, sem):
            pltpu.async_copy(xa_hbm.at[pl.ds(base + o * ch, ch)], buf, sem)

        def wait_load(o, buf, sem):
            pltpu.make_async_copy(
                xa_hbm.at[pl.ds(base + o * ch, ch)], buf, sem).wait()

        def scat(o, buf):
            pltpu.async_copy(buf, acc.at[idx_v.at[o]], ssem, add=True).wait()

        load(0, buf_a, lsem_a)

        def outer(t, carry):
            o = 2 * t
            load(o + 1, buf_b, lsem_b)
            wait_load(o, buf_a, lsem_a)
            scat(o, buf_a)
            load(o + 2, buf_a, lsem_a)
            wait_load(o + 1, buf_b, lsem_b)
            scat(o + 1, buf_b)
            return carry

        lax.fori_loop(0, n_pair, outer, 0)
        wait_load(rows_w - 1, buf_a, lsem_a)
        scat(rows_w - 1, buf_a)
        plsc.subcore_barrier()

        @pl.when(s < n_init)
        def _():
            for t in range(nzch):
                pltpu.sync_copy(acc.at[pl.ds(s * rps + t * zch, zch)], zb_v)
                pltpu.sync_copy(zb_v, out_hbm.at[c, pl.ds(s * rps + t * zch, zch)])

    return k(xa, idx2d)


def kernel(h, m_ij, rbf4, cbf4, sbf4, rbf3, cbf3, rbf_h, idx_i, idx_j,
           idx_swap, params):
    p = params
    n_nodes = h.shape[0]
    de = m_ij.shape[1]

    e_edges = m_ij.shape[0]
    eh = e_edges // 2
    ch = 40
    tile_ce = 2000

    a2, b = _phase_a(m_ij, rbf4, cbf4, sbf4, rbf3, cbf3, rbf_h, p, tile=1600)

    isw = [idx_swap[:eh].reshape(_NW, -1, ch),
           idx_swap[eh:].reshape(_NW, -1, ch)]
    ii = [idx_i[:eh].reshape(_NW, -1, ch), idx_i[eh:].reshape(_NW, -1, ch)]
    ij = [idx_j[:eh].reshape(_NW, -1, ch), idx_j[eh:].reshape(_NW, -1, ch)]

    (bg0,) = _sc_gather_multi([b], [isw[0]])
    (bg1,) = _sc_gather_multi([b], [isw[1]])
    mx0 = _phase_c(a2, bg0, m_ij, de, p, tile=tile_ce, goff=0)
    mx1 = _phase_c(a2, bg1, m_ij, de, p, tile=tile_ce, goff=eh // tile_ce)
    parts0 = _sc_segment_sum(mx0, ii[0], n_nodes)
    parts1 = _sc_segment_sum(mx1, ii[1], n_nodes)
    h_new = _phase_d(parts0, parts1, h, p, tile=2000)
    hi0, hj0 = _sc_gather_multi([h_new, h_new], [ii[0], ij[0]])
    hi1, hj1 = _sc_gather_multi([h_new, h_new], [ii[1], ij[1]])
    m0 = _phase_e(mx0, hi0, hj0, p, tile=tile_ce)
    m1 = _phase_e(mx1, hi1, hj1, p, tile=tile_ce)
    return h_new, jnp.concatenate([m0, m1], axis=0)


# speedup vs baseline: 1.6330x; 1.1274x over previous
import functools

import jax
import jax.numpy as jnp
from jax import lax
from jax.experimental import pallas as pl
from jax.experimental.pallas import tpu as pltpu
from jax.experimental.pallas import tpu_sc as plsc

_INV2 = 1.0 / 2.0 ** 0.5
_INV3 = 1.0 / 3.0 ** 0.5

_NC, _NS = 2, 16
_NW = _NC * _NS
_CH = 80
_K = 5


def _silu(x):
    return x * jax.nn.sigmoid(x)


def _dot(a, b):
    return jnp.dot(a, b, preferred_element_type=jnp.float32)


def _full(w):
    return pl.BlockSpec(w.shape, lambda i: (0,) * w.ndim)


def _phase_a(m_ij, rbf4, cbf4, sbf4, rbf3, cbf3, rbf_h, p, tile):
    E, de = m_ij.shape
    grid = (E // tile,)
    inv_nb = _INV2
    bd = jax.scipy.linalg.block_diag

    w1 = jnp.concatenate([p['q_W_m_rbf'], p['t_W_m_rbf'], p['W_ij']], axis=1)
    g1w = bd(p['q_W_rbf'], p['q_W_rbf'], p['t_W_rbf'], p['t_W_rbf'])
    m2w = bd(p['q_W_m_cbf'], p['q_W_m_cbf'], p['t_W_m_cbf'], p['t_W_m_cbf'])
    g2w = bd(p['q_W_cbf'], p['q_W_cbf'], p['t_W_cbf'], p['t_W_cbf'])
    m3w = bd(p['q_W_m_sbf'], p['q_W_m_sbf'])
    g3w = bd(p['q_W_sbf'], p['q_W_sbf'])
    dirw = bd(p['q_W_dir'], p['t_W_dir'])
    outw = bd(jnp.concatenate([p['q_W_out_ij'], p['q_W_out_ji']], axis=1),
              jnp.concatenate([p['t_W_out_ij'], p['t_W_out_ji']], axis=1))

    def body(m_ref, r4_ref, c4_ref, s4_ref, r3_ref, c3_ref, rh_ref,
             w1r, g1r, m2r, g2r, m3r, g3r, dirr, outr, awrr,
             a_out, b_out):
        m = m_ref[...]
        t0 = _dot(m, w1r[...])
        tq = _silu(t0[:, :de])
        tt = _silu(t0[:, de:2 * de])
        base = t0[:, 2 * de:]
        f32 = lambda x: x.astype(jnp.float32)
        g1in = f32(jnp.concatenate(
            [r4_ref[0], r4_ref[1], r3_ref[0], r3_ref[1]], axis=1))
        u = jnp.concatenate([tq, tq, tt, tt], axis=1) * _dot(g1in, g1r[...])
        u = _silu(_dot(u, m2r[...]))
        g2in = f32(jnp.concatenate(
            [c4_ref[0], c4_ref[1], c3_ref[0], c3_ref[1]], axis=1))
        u = u * _dot(g2in, g2r[...])
        xt = (u[:, 2 * de:3 * de] + u[:, 3 * de:]) * inv_nb
        v = _silu(_dot(u[:, :2 * de], m3r[...]))
        g3in = f32(jnp.concatenate([s4_ref[0], s4_ref[1]], axis=1))
        v = v * _dot(g3in, g3r[...])
        xq = (v[:, :de] + v[:, de:]) * inv_nb
        y = _silu(_dot(jnp.concatenate([xq, xt], axis=1), dirr[...]))
        z = _silu(_dot(y, outr[...]))
        rh_gate = _dot(f32(rh_ref[...]), awrr[...])
        a_out[...] = jnp.concatenate(
            [base + _INV2 * (z[:, :de] + z[:, 2 * de:3 * de]), rh_gate], axis=1)
        b_out[...] = jnp.concatenate([z[:, de:2 * de], z[:, 3 * de:]], axis=1)

    ws = [w1, g1w, m2w, g2w, m3w, g3w, dirw, outw, p['a_W_rbf']]
    edge3 = lambda d: pl.BlockSpec((2, tile, d), lambda i: (0, i, 0))
    return pl.pallas_call(
        body,
        grid=grid,
        in_specs=[pl.BlockSpec((tile, de), lambda i: (i, 0)),
                  edge3(16), edge3(16), edge3(32), edge3(16), edge3(16),
                  pl.BlockSpec((tile, 16), lambda i: (i, 0))]
                 + [_full(w) for w in ws],
        out_specs=[pl.BlockSpec((tile, 2 * de), lambda i: (i, 0)),
                   pl.BlockSpec((tile, 2 * de), lambda i: (i, 0))],
        out_shape=[jax.ShapeDtypeStruct((E, 2 * de), jnp.float32),
                   jax.ShapeDtypeStruct((E, 2 * de), jnp.float32)],
    )(m_ij, rbf4, cbf4, sbf4, rbf3, cbf3, rbf_h, *ws)


def _phase_c(a2, bg, m_ij, de, p, tile, goff):
    eh = bg.shape[0]
    grid = (eh // tile,)

    def body(a_ref, bg_ref, m_ref, bs1, bs2, as1, as2, mx_out):
        a2v = a_ref[...]
        bg = bg_ref[...]
        x = (a2v[:, :de] + _INV2 * (bg[:, :de] + bg[:, de:])) * _INV3
        y = _silu(_dot(x, bs1[...]))
        y = _silu(_dot(y, bs2[...]))
        x = (x + y) * _INV2
        mm = (m_ref[...] + x) * _INV2
        y = _silu(_dot(mm, as1[...]))
        y = _silu(_dot(y, as2[...]))
        mm = (mm + y) * _INV2
        mx_out[...] = jnp.concatenate([mm, mm * a2v[:, de:]], axis=1)

    ws = [p['bs_W1'], p['bs_W2'], p['as_W1'], p['as_W2']]
    specf = pl.BlockSpec((tile, de), lambda i: (i + goff, 0))
    spec2f = pl.BlockSpec((tile, 2 * de), lambda i: (i + goff, 0))
    spec2 = pl.BlockSpec((tile, 2 * de), lambda i: (i, 0))
    return pl.pallas_call(
        body,
        grid=grid,
        in_specs=[spec2f, spec2, specf] + [_full(w) for w in ws],
        out_specs=spec2,
        out_shape=jax.ShapeDtypeStruct((eh, 2 * de), jnp.float32),
    )(a2, bg, m_ij, *ws)


def _phase_d(parts0, parts1, h, p, tile):
    N, da = h.shape
    de = parts0.shape[2] // 2
    grid = (N // tile,)

    def body(p0_ref, p1_ref, h_ref, awd, ar1, ar2, h_out):
        seg = (p0_ref[0, :, de:] + p0_ref[1, :, de:]
               + p1_ref[0, :, de:] + p1_ref[1, :, de:])
        xa = _silu(_dot(seg, awd[...]))
        y = _silu(_dot(xa, ar1[...]))
        y = _silu(_dot(y, ar2[...]))
        xa = (xa + y) * _INV2
        h_out[...] = (h_ref[...] + xa) * _INV2

    ws = [p['a_W_dense'], p['a_res_W1'], p['a_res_W2']]
    pspec = pl.BlockSpec((2, tile, 2 * de), lambda i: (0, i, 0))
    return pl.pallas_call(
        body,
        grid=grid,
        in_specs=[pspec, pspec,
                  pl.BlockSpec((tile, da), lambda i: (i, 0))]
                 + [_full(w) for w in ws],
        out_specs=pl.BlockSpec((tile, da), lambda i: (i, 0)),
        out_shape=jax.ShapeDtypeStruct((N, da), jnp.float32),
    )(parts0, parts1, h, *ws)


def _phase_e(mx, hi, hj, p, tile):
    E = hi.shape[0]
    de = mx.shape[1] // 2
    da = hi.shape[1]
    s_w = p['s_W']
    swi, swj, swm = s_w[:da], s_w[da:2 * da], s_w[2 * da:]
    grid = (E // tile,)

    def body(mx_ref, hi_ref, hj_ref, swi_r, swj_r, swm_r, aa1, aa2, out):
        mm = mx_ref[:, :de]
        m2 = _silu(_dot(hi_ref[...], swi_r[...]) + _dot(hj_ref[...], swj_r[...])
                   + _dot(mm, swm_r[...]))
        y = _silu(_dot(m2, aa1[...]))
        y = _silu(_dot(y, aa2[...]))
        m2 = (m2 + y) * _INV2
        out[...] = (mm + m2) * _INV2

    ws = [swi, swj, swm, p['aa_W1'], p['aa_W2']]
    spec = pl.BlockSpec((tile, de), lambda i: (i, 0))
    spec2 = pl.BlockSpec((tile, 2 * de), lambda i: (i, 0))
    speca = pl.BlockSpec((tile, da), lambda i: (i, 0))
    return pl.pallas_call(
        body,
        grid=grid,
        in_specs=[spec2, speca, speca] + [_full(w) for w in ws],
        out_specs=spec,
        out_shape=jax.ShapeDtypeStruct((E, de), jnp.float32),
    )(mx, hi, hj, *ws)


def _sc_gather_multi(tables, idx2ds):
    n = len(tables)
    w = tables[0].shape[1]
    nw, rows_w, ch = idx2ds[0].shape
    e_total = nw * rows_w * ch
    n_out = rows_w // _K
    mesh = plsc.VectorSubcoreMesh(core_axis_name="c", subcore_axis_name="s")

    @functools.partial(
        pl.kernel, mesh=mesh,
        out_type=[jax.ShapeDtypeStruct((e_total, w), jnp.float32)] * n,
        scratch_types=[pltpu.VMEM((rows_w, ch), jnp.int32)] * n
                      + [pltpu.VMEM((_K * ch, w), jnp.float32)] * n
                      + [pltpu.SemaphoreType.DMA, pltpu.SemaphoreType.DMA],
    )
    def k(*refs):
        tabs = refs[:n]
        idxs = refs[n:2 * n]
        outs = refs[2 * n:3 * n]
        idx_vs = refs[3 * n:4 * n]
        row_vs = refs[4 * n:5 * n]
        gsem, ssem = refs[5 * n:5 * n + 2]
        c = lax.axis_index("c")
        s = lax.axis_index("s")
        wid = s * _NC + c
        base = wid * rows_w * ch
        for t in range(n):
            pltpu.sync_copy(idxs[t].at[wid], idx_vs[t])

        def outer(o, carry):
            cps = []
            for t in range(n):
                for j in range(_K):
                    cps.append(pltpu.async_copy(
                        tabs[t].at[idx_vs[t].at[o * _K + j]],
                        row_vs[t].at[pl.ds(j * ch, ch)], gsem))
            for cp in cps:
                cp.wait()
            sts = []
            for t in range(n):
                sts.append(pltpu.async_copy(
                    row_vs[t], outs[t].at[pl.ds(base + o * _K * ch, _K * ch)],
                    ssem))
            for st in sts:
                st.wait()
            return carry

        lax.fori_loop(0, n_out, outer, 0)

    return k(*tables, *idx2ds)


def _sc_segment_sum(xa, idx2d, n_seg):
    e_total, w = xa.shape
    nw, rows_w, ch = idx2d.shape
    n_pair = (rows_w - 1) // 2
    n_init = 10
    rps = n_seg // n_init
    zch = 40
    nzch = rps // zch
    mesh = plsc.VectorSubcoreMesh(core_axis_name="c", subcore_axis_name="s")

    @functools.partial(
        pl.kernel, mesh=mesh,
        out_type=jax.ShapeDtypeStruct((_NC, n_seg, w), jnp.float32),
        scratch_types=[
            pltpu.VMEM((rows_w, ch), jnp.int32),
            pltpu.VMEM((ch, w), jnp.float32),
            pltpu.VMEM((ch, w), jnp.float32),
            pltpu.VMEM((zch, w), jnp.float32),
            pltpu.VMEM_SHARED((n_seg, w), jnp.float32),
            pltpu.SemaphoreType.DMA,
            pltpu.SemaphoreType.DMA,
            pltpu.SemaphoreType.DMA,
        ],
    )
    def k(xa_hbm, idx_hbm, out_hbm, idx_v, buf_a, buf_b, zb_v, acc,
          lsem_a, lsem_b, ssem):
        c = lax.axis_index("c")
        s = lax.axis_index("s")
        wid = s * _NC + c
        base = wid * rows_w * ch

        def zrow(r, carry):
            for q in range(w // 16):
                zb_v[r, pl.ds(q * 16, 16)] = jnp.zeros((16,), jnp.float32)
            return carry
        lax.fori_loop(0, zch, zrow, 0)

        @pl.when(s < n_init)
        def _():
            for t in range(nzch):
                pltpu.sync_copy(zb_v, acc.at[pl.ds(s * rps + t * zch, zch)])
        pltpu.sync_copy(idx_hbm.at[wid], idx_v)
        plsc.subcore_barrier()

        def load(o, buf, sem):
            pltpu.async_copy(xa_hbm.at[pl.ds(base + o * ch, ch)], buf, sem)

        def wait_load(o, buf, sem):
            pltpu.make_async_copy(
                xa_hbm.at[pl.ds(base + o * ch, ch)], buf, sem).wait()

        def scat(o, buf):
            pltpu.async_copy(buf, acc.at[idx_v.at[o]], ssem, add=True).wait()

        load(0, buf_a, lsem_a)

        def outer(t, carry):
            o = 2 * t
            load(o + 1, buf_b, lsem_b)
            wait_load(o, buf_a, lsem_a)
            scat(o, buf_a)
            load(o + 2, buf_a, lsem_a)
            wait_load(o + 1, buf_b, lsem_b)
            scat(o + 1, buf_b)
            return carry

        lax.fori_loop(0, n_pair, outer, 0)
        wait_load(rows_w - 1, buf_a, lsem_a)
        scat(rows_w - 1, buf_a)
        plsc.subcore_barrier()

        @pl.when(s < n_init)
        def _():
            for t in range(nzch):
                pltpu.sync_copy(acc.at[pl.ds(s * rps + t * zch, zch)], zb_v)
                pltpu.sync_copy(zb_v, out_hbm.at[c, pl.ds(s * rps + t * zch, zch)])

    return k(xa, idx2d)


def kernel(h, m_ij, rbf4, cbf4, sbf4, rbf3, cbf3, rbf_h, idx_i, idx_j,
           idx_swap, params):
    p = params
    n_nodes = h.shape[0]
    de = m_ij.shape[1]

    e_edges = m_ij.shape[0]
    eh = e_edges // 2
    ch = 40
    tile_ce = 2000

    bf = jnp.bfloat16
    a2, b = _phase_a(m_ij, rbf4.astype(bf), cbf4.astype(bf), sbf4.astype(bf),
                     rbf3.astype(bf), cbf3.astype(bf), rbf_h.astype(bf),
                     p, tile=1600)

    isw = [idx_swap[:eh].reshape(_NW, -1, ch),
           idx_swap[eh:].reshape(_NW, -1, ch)]
    ii = [idx_i[:eh].reshape(_NW, -1, ch), idx_i[eh:].reshape(_NW, -1, ch)]
    ij = [idx_j[:eh].reshape(_NW, -1, ch), idx_j[eh:].reshape(_NW, -1, ch)]

    (bg0,) = _sc_gather_multi([b], [isw[0]])
    (bg1,) = _sc_gather_multi([b], [isw[1]])
    mx0 = _phase_c(a2, bg0, m_ij, de, p, tile=tile_ce, goff=0)
    mx1 = _phase_c(a2, bg1, m_ij, de, p, tile=tile_ce, goff=eh // tile_ce)
    parts0 = _sc_segment_sum(mx0, ii[0], n_nodes)
    parts1 = _sc_segment_sum(mx1, ii[1], n_nodes)
    h_new = _phase_d(parts0, parts1, h, p, tile=2000)
    hi0, hj0 = _sc_gather_multi([h_new, h_new], [ii[0], ij[0]])
    hi1, hj1 = _sc_gather_multi([h_new, h_new], [ii[1], ij[1]])
    m0 = _phase_e(mx0, hi0, hj0, p, tile=tile_ce)
    m1 = _phase_e(mx1, hi1, hj1, p, tile=tile_ce)
    return h_new, jnp.concatenate([m0, m1], axis=0)


# speedup vs baseline: 1.7038x; 1.0433x over previous
import functools

import jax
import jax.numpy as jnp
from jax import lax
from jax.experimental import pallas as pl
from jax.experimental.pallas import tpu as pltpu
from jax.experimental.pallas import tpu_sc as plsc

_INV2 = 1.0 / 2.0 ** 0.5
_INV3 = 1.0 / 3.0 ** 0.5

_NC, _NS = 2, 16
_NW = _NC * _NS
_CH = 80
_K = 5


def _silu(x):
    return x * jax.nn.sigmoid(x)


def _dot(a, b):
    return jnp.dot(a, b, preferred_element_type=jnp.float32)


def _full(w):
    return pl.BlockSpec(w.shape, lambda i: (0,) * w.ndim)


def _phase_a(m_ij, rbf4, cbf4, sbf4, rbf3, cbf3, rbf_h, p, tile):
    E, de = m_ij.shape
    grid = (E // tile,)
    inv_nb = _INV2
    bd = jax.scipy.linalg.block_diag

    w1 = jnp.concatenate([p['q_W_m_rbf'], p['t_W_m_rbf'], p['W_ij']], axis=1)
    g1w = bd(p['q_W_rbf'], p['q_W_rbf'], p['t_W_rbf'], p['t_W_rbf'])
    m2w = bd(p['q_W_m_cbf'], p['q_W_m_cbf'], p['t_W_m_cbf'], p['t_W_m_cbf'])
    g2w = bd(p['q_W_cbf'], p['q_W_cbf'], p['t_W_cbf'], p['t_W_cbf'])
    m3w = bd(p['q_W_m_sbf'], p['q_W_m_sbf'])
    g3w = bd(p['q_W_sbf'], p['q_W_sbf'])
    dirw = bd(p['q_W_dir'], p['t_W_dir'])
    outw = bd(jnp.concatenate([p['q_W_out_ij'], p['q_W_out_ji']], axis=1),
              jnp.concatenate([p['t_W_out_ij'], p['t_W_out_ji']], axis=1))

    def body(m_ref, r4_ref, c4_ref, s4_ref, r3_ref, c3_ref, rh_ref,
             w1r, g1r, m2r, g2r, m3r, g3r, dirr, outr, awrr,
             a_out, b_out):
        m = m_ref[...]
        t0 = _dot(m, w1r[...])
        tq = _silu(t0[:, :de])
        tt = _silu(t0[:, de:2 * de])
        base = t0[:, 2 * de:]
        f32 = lambda x: x.astype(jnp.float32)
        g1in = f32(jnp.concatenate(
            [r4_ref[0], r4_ref[1], r3_ref[0], r3_ref[1]], axis=1))
        u = jnp.concatenate([tq, tq, tt, tt], axis=1) * _dot(g1in, g1r[...])
        u = _silu(_dot(u, m2r[...]))
        g2in = f32(jnp.concatenate(
            [c4_ref[0], c4_ref[1], c3_ref[0], c3_ref[1]], axis=1))
        u = u * _dot(g2in, g2r[...])
        xt = (u[:, 2 * de:3 * de] + u[:, 3 * de:]) * inv_nb
        v = _silu(_dot(u[:, :2 * de], m3r[...]))
        g3in = f32(jnp.concatenate([s4_ref[0], s4_ref[1]], axis=1))
        v = v * _dot(g3in, g3r[...])
        xq = (v[:, :de] + v[:, de:]) * inv_nb
        y = _silu(_dot(jnp.concatenate([xq, xt], axis=1), dirr[...]))
        z = _silu(_dot(y, outr[...]))
        rh_gate = _dot(f32(rh_ref[...]), awrr[...])
        a_out[...] = jnp.concatenate(
            [base + _INV2 * (z[:, :de] + z[:, 2 * de:3 * de]), rh_gate], axis=1)
        b_out[...] = jnp.concatenate([z[:, de:2 * de], z[:, 3 * de:]], axis=1)

    ws = [w1, g1w, m2w, g2w, m3w, g3w, dirw, outw, p['a_W_rbf']]
    edge3 = lambda d: pl.BlockSpec((2, tile, d), lambda i: (0, i, 0))
    return pl.pallas_call(
        body,
        grid=grid,
        in_specs=[pl.BlockSpec((tile, de), lambda i: (i, 0)),
                  edge3(16), edge3(16), edge3(32), edge3(16), edge3(16),
                  pl.BlockSpec((tile, 16), lambda i: (i, 0))]
                 + [_full(w) for w in ws],
        out_specs=[pl.BlockSpec((tile, 2 * de), lambda i: (i, 0)),
                   pl.BlockSpec((tile, 2 * de), lambda i: (i, 0))],
        out_shape=[jax.ShapeDtypeStruct((E, 2 * de), jnp.float32),
                   jax.ShapeDtypeStruct((E, 2 * de), jnp.float32)],
    )(m_ij, rbf4, cbf4, sbf4, rbf3, cbf3, rbf_h, *ws)


def _phase_c(a2, bg, m_ij, de, p, tile, goff):
    eh = bg.shape[0]
    grid = (eh // tile,)

    def body(a_ref, bg_ref, m_ref, bs1, bs2, as1, as2, mx_out):
        a2v = a_ref[...]
        bg = bg_ref[...]
        x = (a2v[:, :de] + _INV2 * (bg[:, :de] + bg[:, de:])) * _INV3
        y = _silu(_dot(x, bs1[...]))
        y = _silu(_dot(y, bs2[...]))
        x = (x + y) * _INV2
        mm = (m_ref[...] + x) * _INV2
        y = _silu(_dot(mm, as1[...]))
        y = _silu(_dot(y, as2[...]))
        mm = (mm + y) * _INV2
        mx_out[...] = jnp.concatenate([mm, mm * a2v[:, de:]], axis=1)

    ws = [p['bs_W1'], p['bs_W2'], p['as_W1'], p['as_W2']]
    specf = pl.BlockSpec((tile, de), lambda i: (i + goff, 0))
    spec2f = pl.BlockSpec((tile, 2 * de), lambda i: (i + goff, 0))
    spec2 = pl.BlockSpec((tile, 2 * de), lambda i: (i, 0))
    return pl.pallas_call(
        body,
        grid=grid,
        in_specs=[spec2f, spec2, specf] + [_full(w) for w in ws],
        out_specs=spec2,
        out_shape=jax.ShapeDtypeStruct((eh, 2 * de), jnp.float32),
    )(a2, bg, m_ij, *ws)


def _phase_d(parts0, parts1, h, p, tile):
    N, da = h.shape
    de = parts0.shape[2] // 2
    grid = (N // tile,)

    def body(p0_ref, p1_ref, h_ref, awd, ar1, ar2, h_out):
        seg = (p0_ref[0, :, de:] + p0_ref[1, :, de:]
               + p1_ref[0, :, de:] + p1_ref[1, :, de:])
        xa = _silu(_dot(seg, awd[...]))
        y = _silu(_dot(xa, ar1[...]))
        y = _silu(_dot(y, ar2[...]))
        xa = (xa + y) * _INV2
        h_out[...] = (h_ref[...] + xa) * _INV2

    ws = [p['a_W_dense'], p['a_res_W1'], p['a_res_W2']]
    pspec = pl.BlockSpec((2, tile, 2 * de), lambda i: (0, i, 0))
    return pl.pallas_call(
        body,
        grid=grid,
        in_specs=[pspec, pspec,
                  pl.BlockSpec((tile, da), lambda i: (i, 0))]
                 + [_full(w) for w in ws],
        out_specs=pl.BlockSpec((tile, da), lambda i: (i, 0)),
        out_shape=jax.ShapeDtypeStruct((N, da), jnp.float32),
    )(parts0, parts1, h, *ws)


def _phase_e(mx, hi, hj, p, tile):
    E = hi.shape[0]
    de = mx.shape[1] // 2
    da = hi.shape[1]
    s_w = p['s_W']
    swi, swj, swm = s_w[:da], s_w[da:2 * da], s_w[2 * da:]
    grid = (E // tile,)

    def body(mx_ref, hi_ref, hj_ref, swi_r, swj_r, swm_r, aa1, aa2, out):
        mm = mx_ref[:, :de]
        m2 = _silu(_dot(hi_ref[...], swi_r[...]) + _dot(hj_ref[...], swj_r[...])
                   + _dot(mm, swm_r[...]))
        y = _silu(_dot(m2, aa1[...]))
        y = _silu(_dot(y, aa2[...]))
        m2 = (m2 + y) * _INV2
        out[...] = (mm + m2) * _INV2

    ws = [swi, swj, swm, p['aa_W1'], p['aa_W2']]
    spec = pl.BlockSpec((tile, de), lambda i: (i, 0))
    spec2 = pl.BlockSpec((tile, 2 * de), lambda i: (i, 0))
    speca = pl.BlockSpec((tile, da), lambda i: (i, 0))
    return pl.pallas_call(
        body,
        grid=grid,
        in_specs=[spec2, speca, speca] + [_full(w) for w in ws],
        out_specs=spec,
        out_shape=jax.ShapeDtypeStruct((E, de), jnp.float32),
    )(mx, hi, hj, *ws)


def _sc_gather_multi(tables, idx2ds):
    n = len(tables)
    w = tables[0].shape[1]
    nw, rows_w, ch = idx2ds[0].shape
    e_total = nw * rows_w * ch
    n_out = rows_w // _K
    mesh = plsc.VectorSubcoreMesh(core_axis_name="c", subcore_axis_name="s")

    @functools.partial(
        pl.kernel, mesh=mesh,
        out_type=[jax.ShapeDtypeStruct((e_total, w), jnp.float32)] * n,
        scratch_types=[pltpu.VMEM((rows_w, ch), jnp.int32)] * n
                      + [pltpu.VMEM((_K * ch, w), jnp.float32)] * n
                      + [pltpu.SemaphoreType.DMA, pltpu.SemaphoreType.DMA],
    )
    def k(*refs):
        tabs = refs[:n]
        idxs = refs[n:2 * n]
        outs = refs[2 * n:3 * n]
        idx_vs = refs[3 * n:4 * n]
        row_vs = refs[4 * n:5 * n]
        gsem, ssem = refs[5 * n:5 * n + 2]
        c = lax.axis_index("c")
        s = lax.axis_index("s")
        wid = s * _NC + c
        base = wid * rows_w * ch
        for t in range(n):
            pltpu.sync_copy(idxs[t].at[wid], idx_vs[t])

        def outer(o, carry):
            cps = []
            for t in range(n):
                for j in range(_K):
                    cps.append(pltpu.async_copy(
                        tabs[t].at[idx_vs[t].at[o * _K + j]],
                        row_vs[t].at[pl.ds(j * ch, ch)], gsem))
            for cp in cps:
                cp.wait()
            sts = []
            for t in range(n):
                sts.append(pltpu.async_copy(
                    row_vs[t], outs[t].at[pl.ds(base + o * _K * ch, _K * ch)],
                    ssem))
            for st in sts:
                st.wait()
            return carry

        lax.fori_loop(0, n_out, outer, 0)

    return k(*tables, *idx2ds)


def _sc_segment_sum(xa, idx2d, n_seg):
    e_total, w = xa.shape
    nw, rows_w, ch = idx2d.shape
    n_pair = (rows_w - 1) // 2
    n_init = 10
    rps = n_seg // n_init
    zch = 40
    nzch = rps // zch
    mesh = plsc.VectorSubcoreMesh(core_axis_name="c", subcore_axis_name="s")

    @functools.partial(
        pl.kernel, mesh=mesh,
        out_type=jax.ShapeDtypeStruct((_NC, n_seg, w), jnp.float32),
        scratch_types=[
            pltpu.VMEM((rows_w, ch), jnp.int32),
            pltpu.VMEM((ch, w), jnp.float32),
            pltpu.VMEM((ch, w), jnp.float32),
            pltpu.VMEM((zch, w), jnp.float32),
            pltpu.VMEM_SHARED((n_seg, w), jnp.float32),
            pltpu.SemaphoreType.DMA,
            pltpu.SemaphoreType.DMA,
            pltpu.SemaphoreType.DMA,
        ],
    )
    def k(xa_hbm, idx_hbm, out_hbm, idx_v, buf_a, buf_b, zb_v, acc,
          lsem_a, lsem_b, ssem):
        c = lax.axis_index("c")
        s = lax.axis_index("s")
        wid = s * _NC + c
        base = wid * rows_w * ch

        def zrow(r, carry):
            for q in range(w // 16):
                zb_v[r, pl.ds(q * 16, 16)] = jnp.zeros((16,), jnp.float32)
            return carry
        lax.fori_loop(0, zch, zrow, 0)

        @pl.when(s < n_init)
        def _():
            for t in range(nzch):
                pltpu.sync_copy(zb_v, acc.at[pl.ds(s * rps + t * zch, zch)])
        pltpu.sync_copy(idx_hbm.at[wid], idx_v)
        plsc.subcore_barrier()

        def load(o, buf, sem):
            pltpu.async_copy(xa_hbm.at[pl.ds(base + o * ch, ch)], buf, sem)

        def wait_load(o, buf, sem):
            pltpu.make_async_copy(
                xa_hbm.at[pl.ds(base + o * ch, ch)], buf, sem).wait()

        def scat(o, buf):
            pltpu.async_copy(buf, acc.at[idx_v.at[o]], ssem, add=True).wait()

        load(0, buf_a, lsem_a)

        def outer(t, carry):
            o = 2 * t
            load(o + 1, buf_b, lsem_b)
            wait_load(o, buf_a, lsem_a)
            scat(o, buf_a)
            load(o + 2, buf_a, lsem_a)
            wait_load(o + 1, buf_b, lsem_b)
            scat(o + 1, buf_b)
            return carry

        lax.fori_loop(0, n_pair, outer, 0)
        wait_load(rows_w - 1, buf_a, lsem_a)
        scat(rows_w - 1, buf_a)
        plsc.subcore_barrier()

        @pl.when(s < n_init)
        def _():
            for t in range(nzch):
                pltpu.sync_copy(acc.at[pl.ds(s * rps + t * zch, zch)], zb_v)
                pltpu.sync_copy(zb_v, out_hbm.at[c, pl.ds(s * rps + t * zch, zch)])

    return k(xa, idx2d)


def kernel(h, m_ij, rbf4, cbf4, sbf4, rbf3, cbf3, rbf_h, idx_i, idx_j,
           idx_swap, params):
    p = params
    n_nodes = h.shape[0]
    de = m_ij.shape[1]

    e_edges = m_ij.shape[0]
    eh = e_edges // 2
    ch = 40
    tile_ce = 4000

    bf = jnp.bfloat16
    a2, b = _phase_a(m_ij, rbf4.astype(bf), cbf4.astype(bf), sbf4.astype(bf),
                     rbf3.astype(bf), cbf3.astype(bf), rbf_h.astype(bf),
                     p, tile=3200)

    isw = [idx_swap[:eh].reshape(_NW, -1, ch),
           idx_swap[eh:].reshape(_NW, -1, ch)]
    ii = [idx_i[:eh].reshape(_NW, -1, ch), idx_i[eh:].reshape(_NW, -1, ch)]
    ij = [idx_j[:eh].reshape(_NW, -1, ch), idx_j[eh:].reshape(_NW, -1, ch)]

    (bg0,) = _sc_gather_multi([b], [isw[0]])
    (bg1,) = _sc_gather_multi([b], [isw[1]])
    mx0 = _phase_c(a2, bg0, m_ij, de, p, tile=tile_ce, goff=0)
    mx1 = _phase_c(a2, bg1, m_ij, de, p, tile=tile_ce, goff=eh // tile_ce)
    parts0 = _sc_segment_sum(mx0, ii[0], n_nodes)
    parts1 = _sc_segment_sum(mx1, ii[1], n_nodes)
    h_new = _phase_d(parts0, parts1, h, p, tile=2000)
    hi0, hj0 = _sc_gather_multi([h_new, h_new], [ii[0], ij[0]])
    hi1, hj1 = _sc_gather_multi([h_new, h_new], [ii[1], ij[1]])
    m0 = _phase_e(mx0, hi0, hj0, p, tile=tile_ce)
    m1 = _phase_e(mx1, hi1, hj1, p, tile=tile_ce)
    return h_new, jnp.concatenate([m0, m1], axis=0)


# speedup vs baseline: 1.7285x; 1.0145x over previous
import functools

import jax
import jax.numpy as jnp
from jax import lax
from jax.experimental import pallas as pl
from jax.experimental.pallas import tpu as pltpu
from jax.experimental.pallas import tpu_sc as plsc

_INV2 = 1.0 / 2.0 ** 0.5
_INV3 = 1.0 / 3.0 ** 0.5

_NC, _NS = 2, 16
_NW = _NC * _NS
_CH = 80
_K = 5


def _silu(x):
    return x * jax.nn.sigmoid(x)


def _dot(a, b):
    return jnp.dot(a, b, preferred_element_type=jnp.float32)


def _full(w):
    return pl.BlockSpec(w.shape, lambda i: (0,) * w.ndim)


def _phase_a(m_ij, rbf4, cbf4, sbf4, rbf3, cbf3, rbf_h, p, tile):
    E, de = m_ij.shape
    grid = (E // tile,)
    inv_nb = _INV2
    bd = jax.scipy.linalg.block_diag

    w1 = jnp.concatenate([p['q_W_m_rbf'], p['t_W_m_rbf'], p['W_ij']], axis=1)
    g1w = bd(p['q_W_rbf'], p['q_W_rbf'], p['t_W_rbf'], p['t_W_rbf'])
    m2w = bd(p['q_W_m_cbf'], p['q_W_m_cbf'], p['t_W_m_cbf'], p['t_W_m_cbf'])
    g2w = bd(p['q_W_cbf'], p['q_W_cbf'], p['t_W_cbf'], p['t_W_cbf'])
    m3w = bd(p['q_W_m_sbf'], p['q_W_m_sbf'])
    g3w = bd(p['q_W_sbf'], p['q_W_sbf'])
    dirw = bd(p['q_W_dir'], p['t_W_dir'])
    outw = bd(jnp.concatenate([p['q_W_out_ij'], p['q_W_out_ji']], axis=1),
              jnp.concatenate([p['t_W_out_ij'], p['t_W_out_ji']], axis=1))

    def body(m_ref, r4_ref, c4_ref, s4_ref, r3_ref, c3_ref, rh_ref,
             w1r, g1r, m2r, g2r, m3r, g3r, dirr, outr, awrr,
             a_out, b_out):
        m = m_ref[...].astype(jnp.float32)
        t0 = _dot(m, w1r[...])
        tq = _silu(t0[:, :de])
        tt = _silu(t0[:, de:2 * de])
        base = t0[:, 2 * de:]
        f32 = lambda x: x.astype(jnp.float32)
        g1in = f32(jnp.concatenate(
            [r4_ref[0], r4_ref[1], r3_ref[0], r3_ref[1]], axis=1))
        u = jnp.concatenate([tq, tq, tt, tt], axis=1) * _dot(g1in, g1r[...])
        u = _silu(_dot(u, m2r[...]))
        g2in = f32(jnp.concatenate(
            [c4_ref[0], c4_ref[1], c3_ref[0], c3_ref[1]], axis=1))
        u = u * _dot(g2in, g2r[...])
        xt = (u[:, 2 * de:3 * de] + u[:, 3 * de:]) * inv_nb
        v = _silu(_dot(u[:, :2 * de], m3r[...]))
        g3in = f32(jnp.concatenate([s4_ref[0], s4_ref[1]], axis=1))
        v = v * _dot(g3in, g3r[...])
        xq = (v[:, :de] + v[:, de:]) * inv_nb
        y = _silu(_dot(jnp.concatenate([xq, xt], axis=1), dirr[...]))
        z = _silu(_dot(y, outr[...]))
        rh_gate = _dot(f32(rh_ref[...]), awrr[...])
        a_out[...] = jnp.concatenate(
            [base + _INV2 * (z[:, :de] + z[:, 2 * de:3 * de]), rh_gate], axis=1)
        b_out[...] = jnp.concatenate([z[:, de:2 * de], z[:, 3 * de:]], axis=1)

    ws = [w1, g1w, m2w, g2w, m3w, g3w, dirw, outw, p['a_W_rbf']]
    edge3 = lambda d: pl.BlockSpec((2, tile, d), lambda i: (0, i, 0))
    return pl.pallas_call(
        body,
        grid=grid,
        in_specs=[pl.BlockSpec((tile, de), lambda i: (i, 0)),
                  edge3(16), edge3(16), edge3(32), edge3(16), edge3(16),
                  pl.BlockSpec((tile, 16), lambda i: (i, 0))]
                 + [_full(w) for w in ws],
        out_specs=[pl.BlockSpec((tile, 2 * de), lambda i: (i, 0)),
                   pl.BlockSpec((tile, 2 * de), lambda i: (i, 0))],
        out_shape=[jax.ShapeDtypeStruct((E, 2 * de), jnp.float32),
                   jax.ShapeDtypeStruct((E, 2 * de), jnp.float32)],
    )(m_ij, rbf4, cbf4, sbf4, rbf3, cbf3, rbf_h, *ws)


def _phase_c(a2, bg, m_ij, de, p, tile, goff):
    eh = bg.shape[0]
    grid = (eh // tile,)

    def body(a_ref, bg_ref, m_ref, bs1, bs2, as1, as2, mx_out):
        a2v = a_ref[...]
        bg = bg_ref[...]
        x = (a2v[:, :de] + _INV2 * (bg[:, :de] + bg[:, de:])) * _INV3
        y = _silu(_dot(x, bs1[...]))
        y = _silu(_dot(y, bs2[...]))
        x = (x + y) * _INV2
        mm = (m_ref[...].astype(jnp.float32) + x) * _INV2
        y = _silu(_dot(mm, as1[...]))
        y = _silu(_dot(y, as2[...]))
        mm = (mm + y) * _INV2
        mx_out[...] = jnp.concatenate([mm, mm * a2v[:, de:]], axis=1)

    ws = [p['bs_W1'], p['bs_W2'], p['as_W1'], p['as_W2']]
    specf = pl.BlockSpec((tile, de), lambda i: (i + goff, 0))
    spec2f = pl.BlockSpec((tile, 2 * de), lambda i: (i + goff, 0))
    spec2 = pl.BlockSpec((tile, 2 * de), lambda i: (i, 0))
    return pl.pallas_call(
        body,
        grid=grid,
        in_specs=[spec2f, spec2, specf] + [_full(w) for w in ws],
        out_specs=spec2,
        out_shape=jax.ShapeDtypeStruct((eh, 2 * de), jnp.float32),
    )(a2, bg, m_ij, *ws)


def _phase_d(parts0, parts1, h, p, tile):
    N, da = h.shape
    de = parts0.shape[2] // 2
    grid = (N // tile,)

    def body(p0_ref, p1_ref, h_ref, awd, ar1, ar2, h_out):
        seg = (p0_ref[0, :, de:] + p0_ref[1, :, de:]
               + p1_ref[0, :, de:] + p1_ref[1, :, de:])
        xa = _silu(_dot(seg, awd[...]))
        y = _silu(_dot(xa, ar1[...]))
        y = _silu(_dot(y, ar2[...]))
        xa = (xa + y) * _INV2
        h_out[...] = (h_ref[...] + xa) * _INV2

    ws = [p['a_W_dense'], p['a_res_W1'], p['a_res_W2']]
    pspec = pl.BlockSpec((2, tile, 2 * de), lambda i: (0, i, 0))
    return pl.pallas_call(
        body,
        grid=grid,
        in_specs=[pspec, pspec,
                  pl.BlockSpec((tile, da), lambda i: (i, 0))]
                 + [_full(w) for w in ws],
        out_specs=pl.BlockSpec((tile, da), lambda i: (i, 0)),
        out_shape=jax.ShapeDtypeStruct((N, da), jnp.float32),
    )(parts0, parts1, h, *ws)


def _phase_e(mx, hi, hj, p, tile):
    E = hi.shape[0]
    de = mx.shape[1] // 2
    da = hi.shape[1]
    s_w = p['s_W']
    swi, swj, swm = s_w[:da], s_w[da:2 * da], s_w[2 * da:]
    grid = (E // tile,)

    def body(mx_ref, hi_ref, hj_ref, swi_r, swj_r, swm_r, aa1, aa2, out):
        mm = mx_ref[:, :de]
        m2 = _silu(_dot(hi_ref[...], swi_r[...]) + _dot(hj_ref[...], swj_r[...])
                   + _dot(mm, swm_r[...]))
        y = _silu(_dot(m2, aa1[...]))
        y = _silu(_dot(y, aa2[...]))
        m2 = (m2 + y) * _INV2
        out[...] = (mm + m2) * _INV2

    ws = [swi, swj, swm, p['aa_W1'], p['aa_W2']]
    spec = pl.BlockSpec((tile, de), lambda i: (i, 0))
    spec2 = pl.BlockSpec((tile, 2 * de), lambda i: (i, 0))
    speca = pl.BlockSpec((tile, da), lambda i: (i, 0))
    return pl.pallas_call(
        body,
        grid=grid,
        in_specs=[spec2, speca, speca] + [_full(w) for w in ws],
        out_specs=spec,
        out_shape=jax.ShapeDtypeStruct((E, de), jnp.float32),
    )(mx, hi, hj, *ws)


def _sc_gather_multi(tables, idx2ds):
    n = len(tables)
    w = tables[0].shape[1]
    nw, rows_w, ch = idx2ds[0].shape
    e_total = nw * rows_w * ch
    n_out = rows_w // _K
    mesh = plsc.VectorSubcoreMesh(core_axis_name="c", subcore_axis_name="s")

    @functools.partial(
        pl.kernel, mesh=mesh,
        out_type=[jax.ShapeDtypeStruct((e_total, w), jnp.float32)] * n,
        scratch_types=[pltpu.VMEM((rows_w, ch), jnp.int32)] * n
                      + [pltpu.VMEM((_K * ch, w), jnp.float32)] * n
                      + [pltpu.SemaphoreType.DMA, pltpu.SemaphoreType.DMA],
    )
    def k(*refs):
        tabs = refs[:n]
        idxs = refs[n:2 * n]
        outs = refs[2 * n:3 * n]
        idx_vs = refs[3 * n:4 * n]
        row_vs = refs[4 * n:5 * n]
        gsem, ssem = refs[5 * n:5 * n + 2]
        c = lax.axis_index("c")
        s = lax.axis_index("s")
        wid = s * _NC + c
        base = wid * rows_w * ch
        for t in range(n):
            pltpu.sync_copy(idxs[t].at[wid], idx_vs[t])

        def outer(o, carry):
            cps = []
            for t in range(n):
                for j in range(_K):
                    cps.append(pltpu.async_copy(
                        tabs[t].at[idx_vs[t].at[o * _K + j]],
                        row_vs[t].at[pl.ds(j * ch, ch)], gsem))
            for cp in cps:
                cp.wait()
            sts = []
            for t in range(n):
                sts.append(pltpu.async_copy(
                    row_vs[t], outs[t].at[pl.ds(base + o * _K * ch, _K * ch)],
                    ssem))
            for st in sts:
                st.wait()
            return carry

        lax.fori_loop(0, n_out, outer, 0)

    return k(*tables, *idx2ds)


def _sc_segment_sum(xa, idx2d, n_seg):
    e_total, w = xa.shape
    nw, rows_w, ch = idx2d.shape
    n_pair = (rows_w - 1) // 2
    n_init = 10
    rps = n_seg // n_init
    zch = 40
    nzch = rps // zch
    mesh = plsc.VectorSubcoreMesh(core_axis_name="c", subcore_axis_name="s")

    @functools.partial(
        pl.kernel, mesh=mesh,
        out_type=jax.ShapeDtypeStruct((_NC, n_seg, w), jnp.float32),
        scratch_types=[
            pltpu.VMEM((rows_w, ch), jnp.int32),
            pltpu.VMEM((ch, w), jnp.float32),
            pltpu.VMEM((ch, w), jnp.float32),
            pltpu.VMEM((zch, w), jnp.float32),
            pltpu.VMEM_SHARED((n_seg, w), jnp.float32),
            pltpu.SemaphoreType.DMA,
            pltpu.SemaphoreType.DMA,
            pltpu.SemaphoreType.DMA,
        ],
    )
    def k(xa_hbm, idx_hbm, out_hbm, idx_v, buf_a, buf_b, zb_v, acc,
          lsem_a, lsem_b, ssem):
        c = lax.axis_index("c")
        s = lax.axis_index("s")
        wid = s * _NC + c
        base = wid * rows_w * ch

        def zrow(r, carry):
            for q in range(w // 16):
                zb_v[r, pl.ds(q * 16, 16)] = jnp.zeros((16,), jnp.float32)
            return carry
        lax.fori_loop(0, zch, zrow, 0)

        @pl.when(s < n_init)
        def _():
            for t in range(nzch):
                pltpu.sync_copy(zb_v, acc.at[pl.ds(s * rps + t * zch, zch)])
        pltpu.sync_copy(idx_hbm.at[wid], idx_v)
        plsc.subcore_barrier()

        def load(o, buf, sem):
            pltpu.async_copy(xa_hbm.at[pl.ds(base + o * ch, ch)], buf, sem)

        def wait_load(o, buf, sem):
            pltpu.make_async_copy(
                xa_hbm.at[pl.ds(base + o * ch, ch)], buf, sem).wait()

        def scat(o, buf):
            pltpu.async_copy(buf, acc.at[idx_v.at[o]], ssem, add=True).wait()

        load(0, buf_a, lsem_a)

        def outer(t, carry):
            o = 2 * t
            load(o + 1, buf_b, lsem_b)
            wait_load(o, buf_a, lsem_a)
            scat(o, buf_a)
            load(o + 2, buf_a, lsem_a)
            wait_load(o + 1, buf_b, lsem_b)
            scat(o + 1, buf_b)
            return carry

        lax.fori_loop(0, n_pair, outer, 0)
        wait_load(rows_w - 1, buf_a, lsem_a)
        scat(rows_w - 1, buf_a)
        plsc.subcore_barrier()

        @pl.when(s < n_init)
        def _():
            for t in range(nzch):
                pltpu.sync_copy(acc.at[pl.ds(s * rps + t * zch, zch)], zb_v)
                pltpu.sync_copy(zb_v, out_hbm.at[c, pl.ds(s * rps + t * zch, zch)])

    return k(xa, idx2d)


def kernel(h, m_ij, rbf4, cbf4, sbf4, rbf3, cbf3, rbf_h, idx_i, idx_j,
           idx_swap, params):
    p = params
    n_nodes = h.shape[0]
    de = m_ij.shape[1]

    e_edges = m_ij.shape[0]
    eh = e_edges // 2
    ch = 40
    tile_ce = 4000

    bf = jnp.bfloat16
    m_b = m_ij.astype(bf)
    a2, b = _phase_a(m_b, rbf4.astype(bf), cbf4.astype(bf), sbf4.astype(bf),
                     rbf3.astype(bf), cbf3.astype(bf), rbf_h.astype(bf),
                     p, tile=3200)

    isw = [idx_swap[:eh].reshape(_NW, -1, ch),
           idx_swap[eh:].reshape(_NW, -1, ch)]
    ii = [idx_i[:eh].reshape(_NW, -1, ch), idx_i[eh:].reshape(_NW, -1, ch)]
    ij = [idx_j[:eh].reshape(_NW, -1, ch), idx_j[eh:].reshape(_NW, -1, ch)]

    (bg0,) = _sc_gather_multi([b], [isw[0]])
    (bg1,) = _sc_gather_multi([b], [isw[1]])
    mx0 = _phase_c(a2, bg0, m_b, de, p, tile=tile_ce, goff=0)
    mx1 = _phase_c(a2, bg1, m_b, de, p, tile=tile_ce, goff=eh // tile_ce)
    parts0 = _sc_segment_sum(mx0, ii[0], n_nodes)
    parts1 = _sc_segment_sum(mx1, ii[1], n_nodes)
    h_new = _phase_d(parts0, parts1, h, p, tile=2000)
    hi0, hj0 = _sc_gather_multi([h_new, h_new], [ii[0], ij[0]])
    hi1, hj1 = _sc_gather_multi([h_new, h_new], [ii[1], ij[1]])
    m0 = _phase_e(mx0, hi0, hj0, p, tile=tile_ce)
    m1 = _phase_e(mx1, hi1, hj1, p, tile=tile_ce)
    return h_new, jnp.concatenate([m0, m1], axis=0)
